# baseline (speedup 1.0000x reference)
import functools

import jax
import jax.numpy as jnp
from jax import lax
from jax.experimental import pallas as pl
from jax.experimental.pallas import tpu as pltpu
from jax.experimental.pallas import tpu_sc as plsc


@functools.lru_cache(maxsize=None)
def _make_broadcast_copy(batch: int, seq_len: int, hidden: int):
    info = plsc.get_sparse_core_info()
    num_workers = info.num_cores * info.num_subcores
    assert seq_len % num_workers == 0
    rows_per_w = seq_len // num_workers

    mesh = plsc.VectorSubcoreMesh(core_axis_name="c", subcore_axis_name="s")

    t_rows = ((rows_per_w * 10) // 17 + 7) // 8 * 8
    s_rows = rows_per_w - t_rows

    @functools.partial(
        pl.kernel,
        mesh=mesh,
        out_type=jax.ShapeDtypeStruct((batch, seq_len, hidden), jnp.float32),
        scratch_types=[
            pltpu.VMEM((t_rows, hidden), jnp.float32),
            pltpu.VMEM_SHARED((info.num_subcores, s_rows, hidden),
                              jnp.float32),
            pltpu.SemaphoreType.DMA,
            pltpu.SemaphoreType.DMA,
            pltpu.SemaphoreType.DMA,
            pltpu.SemaphoreType.DMA,
        ],
    )
    def broadcast_copy(table_hbm, out_hbm, tbuf, shared, trsem, twsem,
                       srsem, swsem):
        sid = lax.axis_index("s")
        wid = sid * info.num_cores + lax.axis_index("c")
        base = wid * rows_per_w
        sbase = base + t_rows
        tread = pltpu.async_copy(
            table_hbm.at[pl.ds(base, t_rows)], tbuf, trsem)
        sread = pltpu.async_copy(
            table_hbm.at[pl.ds(sbase, s_rows)], shared.at[sid], srsem)
        tread.wait()
        writes = [
            pltpu.async_copy(
                tbuf, out_hbm.at[b, pl.ds(base, t_rows)], twsem)
            for b in range(batch)
        ]
        sread.wait()
        writes += [
            pltpu.async_copy(
                shared.at[sid], out_hbm.at[b, pl.ds(sbase, s_rows)], swsem)
            for b in range(batch)
        ]
        for w in writes:
            w.wait()

    return broadcast_copy


def kernel(hidden_size, table):
    batch, seq_len, _ = hidden_size.shape
    hidden = table.shape[1]
    return _make_broadcast_copy(batch, seq_len, hidden)(table)
